# baseline (speedup 1.0000x reference)
import jax
import jax.numpy as jnp
from jax import lax
from jax.experimental import pallas as pl
from jax.experimental.pallas import tpu as pltpu
from jax.experimental.pallas import tpu_sc as plsc

N_CLASSES = 1000
CTX_LEN = 77
D_MODEL = 512

NC, NS = 2, 16
NW = NC * NS
B = N_CLASSES * CTX_LEN
K = 120
B0 = 38504
N0, N1 = 21, 21
WPR0, WPR1 = K * N0, K * N1
ST0, ST1 = 2400, 2400
CL0 = B0 - WPR0
CL1 = (B - B0) - WPR1
NMAX = max(N0, N1)


def _gather_body(table_hbm, idx_hbm, out_hbm, idx_v, buf0, buf1, sem0, sem1,
                 osem):
    c = lax.axis_index("c")
    s = lax.axis_index("s")
    nchunk = jnp.where(c == 0, N0, N1)
    wpr = jnp.where(c == 0, WPR0, WPR1)
    base = jnp.where(c == 0,
                     jnp.minimum(s * ST0, CL0),
                     B0 + jnp.minimum(s * ST1, CL1))
    base = pl.multiple_of(base, 8)
    @pl.when(c == 0)
    def _():
        pltpu.sync_copy(idx_hbm.at[pl.ds(base, WPR0)],
                        idx_v.at[pl.ds(0, WPR0)])

    @pl.when(c == 1)
    def _():
        pltpu.sync_copy(idx_hbm.at[pl.ds(base, WPR1)],
                        idx_v.at[pl.ds(0, WPR1)])

    bufs = (buf0, buf1)
    sems = (sem0, sem1)

    pltpu.async_copy(table_hbm.at[idx_v.at[pl.ds(0, K)]], bufs[0], sems[0])

    def body(i, carry):
        slot = lax.rem(i, 2)
        for b in range(2):
            @pl.when(slot == b)
            def _(b=b):
                pltpu.make_async_copy(table_hbm.at[idx_v.at[pl.ds(0, K)]],
                                      bufs[b], sems[b]).wait()

                @pl.when(i + 1 < nchunk)
                def _():
                    off = (i + 1) * K
                    pltpu.async_copy(table_hbm.at[idx_v.at[pl.ds(off, K)]],
                                     bufs[1 - b], sems[1 - b])

                pltpu.async_copy(bufs[b], out_hbm.at[pl.ds(base + i * K, K)],
                                 osem).wait()
        return carry

    lax.fori_loop(0, nchunk, body, 0)


def _sc_gather(table, idx_pad):
    mesh = plsc.VectorSubcoreMesh(core_axis_name="c", subcore_axis_name="s")
    f = pl.kernel(
        _gather_body,
        out_type=jax.ShapeDtypeStruct((B, D_MODEL), jnp.float32),
        mesh=mesh,
        scratch_types=[
            pltpu.VMEM((K * NMAX,), jnp.int32),
            pltpu.VMEM((K, D_MODEL), jnp.float32),
            pltpu.VMEM((K, D_MODEL), jnp.float32),
            pltpu.SemaphoreType.DMA,
            pltpu.SemaphoreType.DMA,
            pltpu.SemaphoreType.DMA,
        ],
        name="sc_embedding_gather",
    )
    return f(table, idx_pad)


def _argmax_body(ids_ref, out_ref):
    ids = ids_ref[...]
    iota = lax.broadcasted_iota(jnp.int32, ids.shape, 1)
    m = jnp.max(ids, axis=1, keepdims=True)
    cand = jnp.where(ids == m, iota, CTX_LEN)
    out_ref[...] = jnp.min(cand, axis=1, keepdims=True)


def _tc_argmax(prompt):
    return pl.pallas_call(
        _argmax_body,
        out_shape=jax.ShapeDtypeStruct((N_CLASSES, 1), jnp.int32),
    )(prompt)


def kernel(prompt, table):
    idx = jnp.swapaxes(prompt, 0, 1).reshape(-1)
    rows = _sc_gather(table, idx)
    embedding = jnp.swapaxes(rows.reshape(CTX_LEN, N_CLASSES, D_MODEL), 0, 1)
    eos = _tc_argmax(prompt).reshape(N_CLASSES)
    return (embedding, eos)


# speedup vs baseline: 1.0111x; 1.0111x over previous
import jax
import jax.numpy as jnp
from jax import lax
from jax.experimental import pallas as pl
from jax.experimental.pallas import tpu as pltpu
from jax.experimental.pallas import tpu_sc as plsc

N_CLASSES = 1000
CTX_LEN = 77
D_MODEL = 512

NC, NS = 2, 16
NW = NC * NS
B = N_CLASSES * CTX_LEN
K = 112
B0 = 38504
N0, N1 = 22, 22
WPR0, WPR1 = K * N0, K * N1
ST0, ST1 = 2408, 2408
CL0 = B0 - WPR0
CL1 = (B - B0) - WPR1
NMAX = max(N0, N1)


def _gather_body(table_hbm, idx_hbm, out_hbm, idx_v, buf0, buf1, sem0, sem1,
                 osem):
    c = lax.axis_index("c")
    s = lax.axis_index("s")
    nchunk = jnp.where(c == 0, N0, N1)
    wpr = jnp.where(c == 0, WPR0, WPR1)
    base = jnp.where(c == 0,
                     jnp.minimum(s * ST0, CL0),
                     B0 + jnp.minimum(s * ST1, CL1))
    base = pl.multiple_of(base, 8)
    @pl.when(c == 0)
    def _():
        pltpu.sync_copy(idx_hbm.at[pl.ds(base, WPR0)],
                        idx_v.at[pl.ds(0, WPR0)])

    @pl.when(c == 1)
    def _():
        pltpu.sync_copy(idx_hbm.at[pl.ds(base, WPR1)],
                        idx_v.at[pl.ds(0, WPR1)])

    bufs = (buf0, buf1)
    sems = (sem0, sem1)

    pltpu.async_copy(table_hbm.at[idx_v.at[pl.ds(0, K)]], bufs[0], sems[0])

    def body(i, carry):
        slot = lax.rem(i, 2)
        for b in range(2):
            @pl.when(slot == b)
            def _(b=b):
                pltpu.make_async_copy(table_hbm.at[idx_v.at[pl.ds(0, K)]],
                                      bufs[b], sems[b]).wait()

                @pl.when(i + 1 < nchunk)
                def _():
                    off = (i + 1) * K
                    pltpu.async_copy(table_hbm.at[idx_v.at[pl.ds(off, K)]],
                                     bufs[1 - b], sems[1 - b])

                pltpu.async_copy(bufs[b], out_hbm.at[pl.ds(base + i * K, K)],
                                 osem).wait()
        return carry

    lax.fori_loop(0, nchunk, body, 0)


def _sc_gather(table, idx_pad):
    mesh = plsc.VectorSubcoreMesh(core_axis_name="c", subcore_axis_name="s")
    f = pl.kernel(
        _gather_body,
        out_type=jax.ShapeDtypeStruct((B, D_MODEL), jnp.float32),
        mesh=mesh,
        scratch_types=[
            pltpu.VMEM((K * NMAX,), jnp.int32),
            pltpu.VMEM((K, D_MODEL), jnp.float32),
            pltpu.VMEM((K, D_MODEL), jnp.float32),
            pltpu.SemaphoreType.DMA,
            pltpu.SemaphoreType.DMA,
            pltpu.SemaphoreType.DMA,
        ],
        name="sc_embedding_gather",
    )
    return f(table, idx_pad)


def _argmax_body(ids_ref, out_ref):
    ids = ids_ref[...]
    iota = lax.broadcasted_iota(jnp.int32, ids.shape, 0)
    m = jnp.max(ids, axis=0, keepdims=True)
    cand = jnp.where(ids == m, iota, CTX_LEN)
    out_ref[...] = jnp.min(cand, axis=0, keepdims=True)


def _tc_argmax(prompt_t):
    return pl.pallas_call(
        _argmax_body,
        out_shape=jax.ShapeDtypeStruct((1, N_CLASSES), jnp.int32),
    )(prompt_t)


def kernel(prompt, table):
    prompt_t = jnp.swapaxes(prompt, 0, 1)
    idx = prompt_t.reshape(-1)
    rows = _sc_gather(table, idx)
    embedding = jnp.swapaxes(rows.reshape(CTX_LEN, N_CLASSES, D_MODEL), 0, 1)
    eos = _tc_argmax(prompt_t).reshape(N_CLASSES)
    return (embedding, eos)


# speedup vs baseline: 1.0299x; 1.0186x over previous
import jax
import jax.numpy as jnp
from jax import lax
from jax.experimental import pallas as pl
from jax.experimental.pallas import tpu as pltpu
from jax.experimental.pallas import tpu_sc as plsc

N_CLASSES = 1000
CTX_LEN = 77
D_MODEL = 512

NC, NS = 2, 16
NW = NC * NS
B = N_CLASSES * CTX_LEN
K = 112
NFULL = 21
KT = 56
WPR = NFULL * K + KT


def _gather_body(table_hbm, idx_hbm, out_hbm, idx_v, buf0, buf1, sem0, sem1,
                 osem):
    c = lax.axis_index("c")
    s = lax.axis_index("s")
    wid = s * NC + c
    base = pl.multiple_of(jnp.minimum(wid * WPR, B - WPR), 8)
    pltpu.sync_copy(idx_hbm.at[pl.ds(base, WPR)], idx_v)

    bufs = (buf0, buf1)
    sems = (sem0, sem1)

    pltpu.async_copy(table_hbm.at[idx_v.at[pl.ds(0, K)]], bufs[0], sems[0])

    def body(i, carry):
        slot = lax.rem(i, 2)
        for b in range(2):
            @pl.when(slot == b)
            def _(b=b):
                pltpu.make_async_copy(table_hbm.at[idx_v.at[pl.ds(0, K)]],
                                      bufs[b], sems[b]).wait()

                @pl.when(i + 1 < NFULL)
                def _():
                    off = (i + 1) * K
                    pltpu.async_copy(table_hbm.at[idx_v.at[pl.ds(off, K)]],
                                     bufs[1 - b], sems[1 - b])

                @pl.when(i + 1 == NFULL)
                def _():
                    pltpu.async_copy(
                        table_hbm.at[idx_v.at[pl.ds(NFULL * K, KT)]],
                        bufs[1 - b].at[pl.ds(0, KT)], sems[1 - b])

                pltpu.async_copy(bufs[b], out_hbm.at[pl.ds(base + i * K, K)],
                                 osem).wait()
        return carry

    lax.fori_loop(0, NFULL, body, 0)

    bt = NFULL % 2
    pltpu.make_async_copy(table_hbm.at[idx_v.at[pl.ds(0, KT)]],
                          bufs[bt].at[pl.ds(0, KT)], sems[bt]).wait()
    pltpu.async_copy(bufs[bt].at[pl.ds(0, KT)],
                     out_hbm.at[pl.ds(base + NFULL * K, KT)], osem).wait()


def _sc_gather(table, idx_pad):
    mesh = plsc.VectorSubcoreMesh(core_axis_name="c", subcore_axis_name="s")
    f = pl.kernel(
        _gather_body,
        out_type=jax.ShapeDtypeStruct((B, D_MODEL), jnp.float32),
        mesh=mesh,
        scratch_types=[
            pltpu.VMEM((WPR,), jnp.int32),
            pltpu.VMEM((K, D_MODEL), jnp.float32),
            pltpu.VMEM((K, D_MODEL), jnp.float32),
            pltpu.SemaphoreType.DMA,
            pltpu.SemaphoreType.DMA,
            pltpu.SemaphoreType.DMA,
        ],
        name="sc_embedding_gather",
    )
    return f(table, idx_pad)


def _argmax_body(ids_ref, out_ref):
    ids = ids_ref[...]
    iota = lax.broadcasted_iota(jnp.int32, ids.shape, 0)
    m = jnp.max(ids, axis=0, keepdims=True)
    cand = jnp.where(ids == m, iota, CTX_LEN)
    out_ref[...] = jnp.min(cand, axis=0, keepdims=True)


def _tc_argmax(prompt_t):
    return pl.pallas_call(
        _argmax_body,
        out_shape=jax.ShapeDtypeStruct((1, N_CLASSES), jnp.int32),
    )(prompt_t)


def kernel(prompt, table):
    prompt_t = jnp.swapaxes(prompt, 0, 1)
    idx = prompt_t.reshape(-1)
    rows = _sc_gather(table, idx)
    embedding = jnp.swapaxes(rows.reshape(CTX_LEN, N_CLASSES, D_MODEL), 0, 1)
    eos = _tc_argmax(prompt_t).reshape(N_CLASSES)
    return (embedding, eos)


# speedup vs baseline: 1.0307x; 1.0008x over previous
import jax
import jax.numpy as jnp
from jax import lax
from jax.experimental import pallas as pl
from jax.experimental.pallas import tpu as pltpu
from jax.experimental.pallas import tpu_sc as plsc

N_CLASSES = 1000
CTX_LEN = 77
D_MODEL = 512

NC, NS = 2, 16
NW = NC * NS
B = N_CLASSES * CTX_LEN
K = 112
NFULL = 21
KT = 56
WPR = NFULL * K + KT


def _gather_body(table_hbm, idx_hbm, out_hbm, idx_v, buf0, buf1, sem0, sem1,
                 osem):
    c = lax.axis_index("c")
    s = lax.axis_index("s")
    wid = s * NC + c
    base = pl.multiple_of(jnp.minimum(wid * WPR, B - WPR), 8)
    bufs = (buf0, buf1)
    sems = (sem0, sem1)

    pltpu.sync_copy(idx_hbm.at[pl.ds(base, K)], idx_v.at[pl.ds(0, K)])
    pltpu.async_copy(table_hbm.at[idx_v.at[pl.ds(0, K)]], bufs[0], sems[0])
    pltpu.sync_copy(idx_hbm.at[pl.ds(base + K, WPR - K)],
                    idx_v.at[pl.ds(K, WPR - K)])

    def body(i, carry):
        slot = lax.rem(i, 2)
        for b in range(2):
            @pl.when(slot == b)
            def _(b=b):
                pltpu.make_async_copy(table_hbm.at[idx_v.at[pl.ds(0, K)]],
                                      bufs[b], sems[b]).wait()

                @pl.when(i + 1 < NFULL)
                def _():
                    off = (i + 1) * K
                    pltpu.async_copy(table_hbm.at[idx_v.at[pl.ds(off, K)]],
                                     bufs[1 - b], sems[1 - b])

                @pl.when(i + 1 == NFULL)
                def _():
                    pltpu.async_copy(
                        table_hbm.at[idx_v.at[pl.ds(NFULL * K, KT)]],
                        bufs[1 - b].at[pl.ds(0, KT)], sems[1 - b])

                pltpu.async_copy(bufs[b], out_hbm.at[pl.ds(base + i * K, K)],
                                 osem).wait()
        return carry

    lax.fori_loop(0, NFULL, body, 0)

    bt = NFULL % 2
    pltpu.make_async_copy(table_hbm.at[idx_v.at[pl.ds(0, KT)]],
                          bufs[bt].at[pl.ds(0, KT)], sems[bt]).wait()
    pltpu.async_copy(bufs[bt].at[pl.ds(0, KT)],
                     out_hbm.at[pl.ds(base + NFULL * K, KT)], osem).wait()


def _sc_gather(table, idx_pad):
    mesh = plsc.VectorSubcoreMesh(core_axis_name="c", subcore_axis_name="s")
    f = pl.kernel(
        _gather_body,
        out_type=jax.ShapeDtypeStruct((B, D_MODEL), jnp.float32),
        mesh=mesh,
        scratch_types=[
            pltpu.VMEM((WPR,), jnp.int32),
            pltpu.VMEM((K, D_MODEL), jnp.float32),
            pltpu.VMEM((K, D_MODEL), jnp.float32),
            pltpu.SemaphoreType.DMA,
            pltpu.SemaphoreType.DMA,
            pltpu.SemaphoreType.DMA,
        ],
        name="sc_embedding_gather",
    )
    return f(table, idx_pad)


def _argmax_body(ids_ref, out_ref):
    ids = ids_ref[...]
    iota = lax.broadcasted_iota(jnp.int32, ids.shape, 0)
    m = jnp.max(ids, axis=0, keepdims=True)
    cand = jnp.where(ids == m, iota, CTX_LEN)
    out_ref[...] = jnp.min(cand, axis=0, keepdims=True)


def _tc_argmax(prompt_t):
    return pl.pallas_call(
        _argmax_body,
        out_shape=jax.ShapeDtypeStruct((1, N_CLASSES), jnp.int32),
    )(prompt_t)


def kernel(prompt, table):
    prompt_t = jnp.swapaxes(prompt, 0, 1)
    idx = prompt_t.reshape(-1)
    rows = _sc_gather(table, idx)
    embedding = jnp.swapaxes(rows.reshape(CTX_LEN, N_CLASSES, D_MODEL), 0, 1)
    eos = _tc_argmax(prompt_t).reshape(N_CLASSES)
    return (embedding, eos)
